# baseline (device time: 202640 ns/iter reference)
import jax
import jax.numpy as jnp
from jax import lax
from jax.experimental import pallas as pl
from jax.experimental.pallas import tpu as pltpu

N_Y = 4
B, S, D = 2, 512, 2048
H, DH, DR = 16, 128, 32
DC_SH = 128
BS = B * S
NCH = 4
CH = D // NCH
QCH = 128
NQ = D // QCH
KC = 512
NKC = D // KC
RW = 256
SCALE = (DH + DR) ** -0.5


def _fused_main(x2, Wdkv, Wuk, Wuv, Wq, Wqr, Wkr):

    def body(x_ref, wdkv_ref, wuk_ref, wuv_ref, wq_hbm, wqr_ref, wkr_ref,
             k_ref, v_ref, q_ref, qrt_ref, kr_ref,
             c4_ref, wuk4_ref, wuv4_ref, wq_buf, send_sems, recv_sems,
             wq_sems):
        xi = lax.axis_index("x")
        my = lax.axis_index("y")
        zi = lax.axis_index("z")
        left = lax.rem(my + N_Y - 1, N_Y)
        right = lax.rem(my + 1, N_Y)

        barrier = pltpu.get_barrier_semaphore()
        pl.semaphore_signal(barrier, inc=1, device_id=(xi, left, zi),
                            device_id_type=pl.DeviceIdType.MESH)
        pl.semaphore_signal(barrier, inc=1, device_id=(xi, right, zi),
                            device_id_type=pl.DeviceIdType.MESH)
        pl.semaphore_wait(barrier, 2)

        def wq_copy(j):
            return pltpu.make_async_copy(
                wq_hbm.at[:, pl.ds(j * QCH, QCH)],
                wq_buf.at[j % 2],
                wq_sems.at[j % 2])

        wq_copy(0).start()

        for r in range(BS // RW):
            xr = x_ref[pl.ds(r * RW, RW), :]
            c4_ref[my, :, pl.ds(r * RW, RW)] = lax.dot_general(
                wdkv_ref[:, :], xr, (((0,), (1,)), ((), ())),
                preferred_element_type=jnp.float32)
        wuk4_ref[my] = wuk_ref[:, :]
        wuv4_ref[my] = wuv_ref[:, :]

        def start_hop(h):
            slot = lax.rem(my + N_Y - h, N_Y)
            rdmas = []
            for t, ref in enumerate((c4_ref, wuk4_ref, wuv4_ref)):
                src = ref.at[slot]
                rdma = pltpu.make_async_remote_copy(
                    src_ref=src,
                    dst_ref=src,
                    send_sem=send_sems.at[t, h],
                    recv_sem=recv_sems.at[t, h],
                    device_id=(xi, right, zi),
                    device_id_type=pl.DeviceIdType.MESH,
                )
                rdma.start()
                rdmas.append(rdma)
            return rdmas

        def acc_kv(slot, first=False):
            c_t = c4_ref[slot]
            for jc in range(NCH):
                cols = pl.ds(jc * CH, CH)
                for out_ref, w4_ref in ((k_ref, wuk4_ref), (v_ref, wuv4_ref)):
                    contrib = lax.dot_general(
                        c_t, w4_ref[slot, :, cols], (((0,), (0,)), ((), ())),
                        preferred_element_type=jnp.float32)
                    if first:
                        out_ref[:, cols] = contrib
                    else:
                        out_ref[:, cols] += contrib

        def do_q(j):
            wq_copy(j).wait()
            if j + 1 < NQ:
                wq_copy(j + 1).start()
            buf = wq_buf.at[j % 2]
            acc = lax.dot_general(
                x_ref[:, pl.ds(0, KC)], buf[pl.ds(0, KC), :],
                (((1,), (0,)), ((), ())),
                preferred_element_type=jnp.float32)
            for kc in range(1, NKC):
                acc += lax.dot_general(
                    x_ref[:, pl.ds(kc * KC, KC)], buf[pl.ds(kc * KC, KC), :],
                    (((1,), (0,)), ((), ())),
                    preferred_element_type=jnp.float32)
            q_ref[:, pl.ds(j * QCH, QCH)] = acc

        def do_qrt(r):
            rows = pl.ds(r * RW, RW)
            acc = lax.dot_general(
                wqr_ref[pl.ds(0, KC), :], x_ref[rows, pl.ds(0, KC)],
                (((0,), (1,)), ((), ())),
                preferred_element_type=jnp.float32)
            for kc in range(1, NKC):
                acc += lax.dot_general(
                    wqr_ref[pl.ds(kc * KC, KC), :],
                    x_ref[rows, pl.ds(kc * KC, KC)],
                    (((0,), (1,)), ((), ())),
                    preferred_element_type=jnp.float32)
            qrt_ref[:, pl.ds(r * RW, RW)] = acc

        def do_kr():
            acc = jnp.dot(x_ref[:, pl.ds(0, KC)], wkr_ref[pl.ds(0, KC), :],
                          preferred_element_type=jnp.float32)
            for kc in range(1, NKC):
                acc += jnp.dot(x_ref[:, pl.ds(kc * KC, KC)],
                               wkr_ref[pl.ds(kc * KC, KC), :],
                               preferred_element_type=jnp.float32)
            kr_ref[:, :] = acc

        rdmas = start_hop(0)
        acc_kv(my, first=True)
        for j in range(0, 5):
            do_q(j)
        for r in rdmas:
            r.wait()

        rdmas = start_hop(1)
        acc_kv(lax.rem(my + N_Y - 1, N_Y))
        for j in range(5, 10):
            do_q(j)
        for r in rdmas:
            r.wait()

        rdmas = start_hop(2)
        acc_kv(lax.rem(my + N_Y - 2, N_Y))
        for j in range(10, NQ):
            do_q(j)
        do_kr()
        for r in rdmas:
            r.wait()

        acc_kv(lax.rem(my + 1, N_Y))
        for r in range(BS // RW):
            do_qrt(r)

    return pl.pallas_call(
        body,
        in_specs=[
            pl.BlockSpec(memory_space=pltpu.VMEM),
            pl.BlockSpec(memory_space=pltpu.VMEM),
            pl.BlockSpec(memory_space=pltpu.VMEM),
            pl.BlockSpec(memory_space=pltpu.VMEM),
            pl.BlockSpec(memory_space=pl.ANY),
            pl.BlockSpec(memory_space=pltpu.VMEM),
            pl.BlockSpec(memory_space=pltpu.VMEM),
        ],
        out_specs=[pl.BlockSpec(memory_space=pltpu.VMEM)] * 5,
        out_shape=[
            jax.ShapeDtypeStruct((BS, D), jnp.float32),
            jax.ShapeDtypeStruct((BS, D), jnp.float32),
            jax.ShapeDtypeStruct((BS, D), jnp.float32),
            jax.ShapeDtypeStruct((H * DR, BS), jnp.float32),
            jax.ShapeDtypeStruct((BS, DR), jnp.float32),
        ],
        scratch_shapes=[
            pltpu.VMEM((N_Y, DC_SH, BS), jnp.float32),
            pltpu.VMEM((N_Y, DC_SH, D), jnp.float32),
            pltpu.VMEM((N_Y, DC_SH, D), jnp.float32),
            pltpu.VMEM((2, D, QCH), jnp.float32),
            pltpu.SemaphoreType.DMA((3, N_Y - 1)),
            pltpu.SemaphoreType.DMA((3, N_Y - 1)),
            pltpu.SemaphoreType.DMA((2,)),
        ],
        compiler_params=pltpu.CompilerParams(
            collective_id=0, vmem_limit_bytes=62 * 1024 * 1024),
    )(x2, Wdkv, Wuk, Wuv, Wq, Wqr, Wkr)


def _attention_out(Q, K, V, QrT, Kr, Wo):

    def body(q_ref, k_ref, v_ref, qr_ref, kr_ref, wo_ref, o_ref):
        h = pl.program_id(1)
        q = q_ref[:, :]
        k = k_ref[:, :]
        v = v_ref[:, :]
        qr_t = qr_ref[:, :]
        kr = kr_ref[:, :]
        s = lax.dot_general(q, k, (((1,), (1,)), ((), ())),
                            preferred_element_type=jnp.float32)
        s = s + lax.dot_general(qr_t, kr, (((0,), (1,)), ((), ())),
                                preferred_element_type=jnp.float32)
        s = s * SCALE
        m = jnp.max(s, axis=-1, keepdims=True)
        p = jnp.exp(s - m)
        p = p / jnp.sum(p, axis=-1, keepdims=True)
        o_h = jnp.dot(p, v, preferred_element_type=jnp.float32)
        contrib = jnp.dot(o_h, wo_ref[:, :],
                          preferred_element_type=jnp.float32)

        @pl.when(h == 0)
        def _():
            o_ref[:, :] = contrib

        @pl.when(h != 0)
        def _():
            o_ref[:, :] += contrib

    return pl.pallas_call(
        body,
        grid=(B, H),
        in_specs=[
            pl.BlockSpec((S, DH), lambda b, h: (b, h)),
            pl.BlockSpec((S, DH), lambda b, h: (b, h)),
            pl.BlockSpec((S, DH), lambda b, h: (b, h)),
            pl.BlockSpec((DR, S), lambda b, h: (h, b)),
            pl.BlockSpec((S, DR), lambda b, h: (b, 0)),
            pl.BlockSpec((DH, D), lambda b, h: (h, 0)),
        ],
        out_specs=pl.BlockSpec((S, D), lambda b, h: (b, 0)),
        out_shape=jax.ShapeDtypeStruct((BS, D), jnp.float32),
    )(Q, K, V, QrT, Kr, Wo)


def kernel(x, Wdkv, Wuk, Wuv, Wq, Wqr, Wkr, Wo):
    x2 = x.reshape(BS, D)
    K, V, Q, QrT, Kr = _fused_main(x2, Wdkv, Wuk, Wuv, Wq, Wqr, Wkr)
    out = _attention_out(Q, K, V, QrT, Kr, Wo)
    return out.reshape(B, S, D)


# device time: 136903 ns/iter; 1.4802x vs baseline; 1.4802x over previous
import jax
import jax.numpy as jnp
from jax import lax
from jax.experimental import pallas as pl
from jax.experimental.pallas import tpu as pltpu

N_Y = 4
B, S, D = 2, 512, 2048
H, DH, DR = 16, 128, 32
DC_SH = 128
BS = B * S
NCH = 4
CH = D // NCH
QCH = 128
NQ = D // QCH
KC = 512
NKC = D // KC
RW = 256
SCALE = (DH + DR) ** -0.5


def _mm(a, b, bn=1024):
    m, k = a.shape
    _, n = b.shape
    bn = min(bn, n)

    def body(a_ref, b_ref, o_ref):
        o_ref[:, :] = jnp.dot(a_ref[:, :], b_ref[:, :],
                              preferred_element_type=jnp.float32)

    return pl.pallas_call(
        body,
        grid=(n // bn,),
        in_specs=[
            pl.BlockSpec((m, k), lambda j: (0, 0)),
            pl.BlockSpec((k, bn), lambda j: (0, j)),
        ],
        out_specs=pl.BlockSpec((m, bn), lambda j: (0, j)),
        out_shape=jax.ShapeDtypeStruct((m, n), jnp.float32),
    )(a, b)


def _fused_main(x2, Wdkv, Wuk, Wuv, Wq, Wqr, Wkr):

    def body(x_ref, wdkv_ref, wuk_ref, wuv_ref, wq_hbm, wqr_ref, wkr_ref,
             k_ref, v_ref, q_ref, qrt_ref, kr_ref,
             c4_ref, wuk4_ref, wuv4_ref, wq_buf, send_sems, recv_sems,
             wq_sems):
        xi = lax.axis_index("x")
        my = lax.axis_index("y")
        zi = lax.axis_index("z")
        left = lax.rem(my + N_Y - 1, N_Y)
        right = lax.rem(my + 1, N_Y)

        barrier = pltpu.get_barrier_semaphore()
        pl.semaphore_signal(barrier, inc=1, device_id=(xi, left, zi),
                            device_id_type=pl.DeviceIdType.MESH)
        pl.semaphore_signal(barrier, inc=1, device_id=(xi, right, zi),
                            device_id_type=pl.DeviceIdType.MESH)
        pl.semaphore_wait(barrier, 2)

        def wq_copy(j):
            return pltpu.make_async_copy(
                wq_hbm.at[:, pl.ds(j * QCH, QCH)],
                wq_buf.at[j % 2],
                wq_sems.at[j % 2])

        wq_copy(0).start()

        for r in range(BS // RW):
            xr = x_ref[pl.ds(r * RW, RW), :]
            c4_ref[my, :, pl.ds(r * RW, RW)] = lax.dot_general(
                wdkv_ref[:, :], xr, (((0,), (1,)), ((), ())),
                preferred_element_type=jnp.float32).astype(jnp.bfloat16)
        wuk4_ref[my] = wuk_ref[:, :].astype(jnp.bfloat16)
        wuv4_ref[my] = wuv_ref[:, :].astype(jnp.bfloat16)

        def start_hop(h):
            slot = lax.rem(my + N_Y - h, N_Y)
            rdmas = []
            for t, ref in enumerate((c4_ref, wuk4_ref, wuv4_ref)):
                src = ref.at[slot]
                rdma = pltpu.make_async_remote_copy(
                    src_ref=src,
                    dst_ref=src,
                    send_sem=send_sems.at[t, h],
                    recv_sem=recv_sems.at[t, h],
                    device_id=(xi, right, zi),
                    device_id_type=pl.DeviceIdType.MESH,
                )
                rdma.start()
                rdmas.append(rdma)
            return rdmas

        def acc_kv(slot, first=False):
            c_t = c4_ref[slot]
            for jc in range(NCH):
                cols = pl.ds(jc * CH, CH)
                for out_ref, w4_ref in ((k_ref, wuk4_ref), (v_ref, wuv4_ref)):
                    contrib = lax.dot_general(
                        c_t, w4_ref[slot, :, cols], (((0,), (0,)), ((), ())),
                        preferred_element_type=jnp.float32)
                    if first:
                        out_ref[:, cols] = contrib
                    else:
                        out_ref[:, cols] += contrib

        def do_q(j):
            wq_copy(j).wait()
            if j + 1 < NQ:
                wq_copy(j + 1).start()
            buf = wq_buf.at[j % 2]
            acc = lax.dot_general(
                x_ref[:, pl.ds(0, KC)], buf[pl.ds(0, KC), :],
                (((1,), (0,)), ((), ())),
                preferred_element_type=jnp.float32)
            for kc in range(1, NKC):
                acc += lax.dot_general(
                    x_ref[:, pl.ds(kc * KC, KC)], buf[pl.ds(kc * KC, KC), :],
                    (((1,), (0,)), ((), ())),
                    preferred_element_type=jnp.float32)
            q_ref[:, pl.ds(j * QCH, QCH)] = acc

        def do_qrt(r):
            rows = pl.ds(r * RW, RW)
            acc = lax.dot_general(
                wqr_ref[pl.ds(0, KC), :], x_ref[rows, pl.ds(0, KC)],
                (((0,), (1,)), ((), ())),
                preferred_element_type=jnp.float32)
            for kc in range(1, NKC):
                acc += lax.dot_general(
                    wqr_ref[pl.ds(kc * KC, KC), :],
                    x_ref[rows, pl.ds(kc * KC, KC)],
                    (((0,), (1,)), ((), ())),
                    preferred_element_type=jnp.float32)
            qrt_ref[:, pl.ds(r * RW, RW)] = acc

        def do_kr():
            acc = jnp.dot(x_ref[:, pl.ds(0, KC)], wkr_ref[pl.ds(0, KC), :],
                          preferred_element_type=jnp.float32)
            for kc in range(1, NKC):
                acc += jnp.dot(x_ref[:, pl.ds(kc * KC, KC)],
                               wkr_ref[pl.ds(kc * KC, KC), :],
                               preferred_element_type=jnp.float32)
            kr_ref[:, :] = acc

        rdmas = start_hop(0)
        acc_kv(my, first=True)
        for j in range(0, 5):
            do_q(j)
        for r in rdmas:
            r.wait()

        rdmas = start_hop(1)
        acc_kv(lax.rem(my + N_Y - 1, N_Y))
        for j in range(5, 10):
            do_q(j)
        do_qrt(0)
        do_qrt(1)
        for r in rdmas:
            r.wait()

        rdmas = start_hop(2)
        acc_kv(lax.rem(my + N_Y - 2, N_Y))
        for j in range(10, NQ):
            do_q(j)
        do_qrt(2)
        do_qrt(3)
        do_kr()
        for r in rdmas:
            r.wait()

        acc_kv(lax.rem(my + 1, N_Y))

    return pl.pallas_call(
        body,
        in_specs=[
            pl.BlockSpec(memory_space=pltpu.VMEM),
            pl.BlockSpec(memory_space=pltpu.VMEM),
            pl.BlockSpec(memory_space=pltpu.VMEM),
            pl.BlockSpec(memory_space=pltpu.VMEM),
            pl.BlockSpec(memory_space=pl.ANY),
            pl.BlockSpec(memory_space=pltpu.VMEM),
            pl.BlockSpec(memory_space=pltpu.VMEM),
        ],
        out_specs=[pl.BlockSpec(memory_space=pltpu.VMEM)] * 5,
        out_shape=[
            jax.ShapeDtypeStruct((BS, D), jnp.float32),
            jax.ShapeDtypeStruct((BS, D), jnp.float32),
            jax.ShapeDtypeStruct((BS, D), jnp.float32),
            jax.ShapeDtypeStruct((H * DR, BS), jnp.float32),
            jax.ShapeDtypeStruct((BS, DR), jnp.float32),
        ],
        scratch_shapes=[
            pltpu.VMEM((N_Y, DC_SH, BS), jnp.bfloat16),
            pltpu.VMEM((N_Y, DC_SH, D), jnp.bfloat16),
            pltpu.VMEM((N_Y, DC_SH, D), jnp.bfloat16),
            pltpu.VMEM((2, D, QCH), jnp.float32),
            pltpu.SemaphoreType.DMA((3, N_Y - 1)),
            pltpu.SemaphoreType.DMA((3, N_Y - 1)),
            pltpu.SemaphoreType.DMA((2,)),
        ],
        compiler_params=pltpu.CompilerParams(
            collective_id=0, vmem_limit_bytes=62 * 1024 * 1024),
    )(x2, Wdkv, Wuk, Wuv, Wq, Wqr, Wkr)


def _attention(Q, K, V, QrT, Kr):

    def body(q_ref, k_ref, v_ref, qr_ref, kr_ref, o_ref):
        q = q_ref[:, :]
        k = k_ref[:, :]
        v = v_ref[:, :]
        qr_t = qr_ref[:, :]
        kr = kr_ref[:, :]
        s = lax.dot_general(q, k, (((1,), (1,)), ((), ())),
                            preferred_element_type=jnp.float32)
        s = s + lax.dot_general(qr_t, kr, (((0,), (1,)), ((), ())),
                                preferred_element_type=jnp.float32)
        s = s * SCALE
        m = jnp.max(s, axis=-1, keepdims=True)
        p = jnp.exp(s - m)
        p = p / jnp.sum(p, axis=-1, keepdims=True)
        o_ref[:, :] = jnp.dot(p, v, preferred_element_type=jnp.float32)

    return pl.pallas_call(
        body,
        grid=(B, H),
        in_specs=[
            pl.BlockSpec((S, DH), lambda b, h: (b, h)),
            pl.BlockSpec((S, DH), lambda b, h: (b, h)),
            pl.BlockSpec((S, DH), lambda b, h: (b, h)),
            pl.BlockSpec((DR, S), lambda b, h: (h, b)),
            pl.BlockSpec((S, DR), lambda b, h: (b, 0)),
        ],
        out_specs=pl.BlockSpec((S, DH), lambda b, h: (b, h)),
        out_shape=jax.ShapeDtypeStruct((BS, D), jnp.float32),
    )(Q, K, V, QrT, Kr)


def kernel(x, Wdkv, Wuk, Wuv, Wq, Wqr, Wkr, Wo):
    x2 = x.reshape(BS, D)
    K, V, Q, QrT, Kr = _fused_main(x2, Wdkv, Wuk, Wuv, Wq, Wqr, Wkr)
    O = _attention(Q, K, V, QrT, Kr)
    out = _mm(O, Wo)
    return out.reshape(B, S, D)
